# baseline (speedup 1.0000x reference)
import functools

import jax
import jax.numpy as jnp
from jax import lax
from jax.experimental import pallas as pl
from jax.experimental.pallas import tpu as pltpu
from jax.experimental.pallas import tpu_sc as plsc

UP_TO = 512
DIM = 64
D = DIM * DIM

_NC = 2
_NS = 16
_NW = _NC * _NS

_MW = 2048


def _build_table_kernel(prim_ref, table_ref):
    b0 = prim_ref[0]
    b1 = prim_ref[1]
    table_ref[pl.ds(0, 2)] = prim_ref[pl.ds(0, 2)]
    t2 = jnp.dot(b1, b0, preferred_element_type=jnp.float32)
    table_ref[pl.ds(2, 1)] = t2[None]
    filled = 3
    while filled < UP_TO:
        p0 = (filled + 1) // 2
        s = filled - p0
        parents = table_ref[pl.ds(p0, s)]
        pf = parents.reshape(s * DIM, DIM)
        c1 = jnp.dot(pf, b1, preferred_element_type=jnp.float32)
        c0 = jnp.dot(pf, b0, preferred_element_type=jnp.float32)
        ch = jnp.stack(
            [c1.reshape(s, DIM, DIM), c0.reshape(s, DIM, DIM)], axis=1
        ).reshape(2 * s, DIM, DIM)
        start = 2 * p0 - 1
        cnt = min(2 * s, UP_TO - start)
        table_ref[pl.ds(start, cnt)] = ch[:cnt] if cnt != 2 * s else ch
        filled = start + cnt


def _build_table(primitives):
    return pl.pallas_call(
        _build_table_kernel,
        out_shape=jax.ShapeDtypeStruct((UP_TO, DIM, DIM), jnp.float32),
    )(primitives)


_TC = 128


def _transpose_kernel(in_ref, out_ref):
    out_ref[...] = jnp.transpose(in_ref[...], (1, 0))


def _transpose_table(table2d):
    return pl.pallas_call(
        _transpose_kernel,
        grid=(D // _TC,),
        in_specs=[pl.BlockSpec((UP_TO, _TC), lambda j: (0, j))],
        out_specs=pl.BlockSpec((_TC, UP_TO), lambda j: (j, 0)),
        out_shape=jax.ShapeDtypeStruct((D, UP_TO), jnp.float32),
    )(table2d)


def _make_tgather(n_lookups):
    rpw = D // _NW
    ntr = rpw // 8
    nmc = n_lookups // _MW
    mesh = plsc.VectorSubcoreMesh(core_axis_name="c", subcore_axis_name="s")

    @functools.partial(
        pl.kernel,
        out_type=jax.ShapeDtypeStruct((D, n_lookups), jnp.float32),
        mesh=mesh,
        compiler_params=pltpu.CompilerParams(needs_layout_passes=False),
        scratch_types=[
            pltpu.VMEM((rpw, UP_TO), jnp.float32),
            pltpu.VMEM((n_lookups,), jnp.int32),
            pltpu.VMEM((8, _MW), jnp.float32),
            pltpu.VMEM((8, _MW), jnp.float32),
            pltpu.SemaphoreType.DMA,
            pltpu.SemaphoreType.DMA,
        ],
    )
    def tgather_kernel(tt_hbm, idx_hbm, out_hbm, tab_v, idx_v, buf0, buf1,
                       sem0, sem1):
        wid = lax.axis_index("s") * _NC + lax.axis_index("c")
        r0 = wid * rpw
        pltpu.sync_copy(tt_hbm.at[pl.ds(r0, rpw)], tab_v)
        pltpu.sync_copy(idx_hbm, idx_v)

        bufs = (buf0, buf1)
        sems = (sem0, sem1)
        nblk = ntr * nmc

        @pl.loop(0, nblk, step=2)
        def _(b):
            for k in range(2):
                bb = b + k
                buf = bufs[k]
                sem = sems[k]
                tr = bb // nmc
                mc = bb % nmc
                dst = out_hbm.at[
                    pl.ds(r0 + tr * 8, 8), pl.ds(mc * _MW, _MW)
                ]

                @pl.when(bb >= 2)
                def _():
                    pltpu.make_async_copy(buf, dst, sem).wait()

                m0 = mc * _MW
                rvecs = [
                    jnp.full((16,), tr * 8 + s, jnp.int32) for s in range(8)
                ]
                @plsc.parallel_loop(0, _MW // 16, unroll=16)
                def _(jv):
                    mvec = idx_v[pl.ds(m0 + jv * 16, 16)]
                    for s in range(8):
                        vals = plsc.load_gather(tab_v, [rvecs[s], mvec])
                        buf[s, pl.ds(jv * 16, 16)] = vals

                pltpu.async_copy(buf, dst, sem)

        for k in range(2):
            last = nblk - 2 + k
            tr = last // nmc
            mc = last % nmc
            dst = out_hbm.at[pl.ds(r0 + tr * 8, 8), pl.ds(mc * _MW, _MW)]
            pltpu.make_async_copy(bufs[k], dst, sems[k]).wait()

    return tgather_kernel


def kernel(primitives, node_positions):
    n = node_positions.shape[0]
    table = _build_table(primitives)
    tt = table.reshape(UP_TO, D).T
    idx = (node_positions - 1).astype(jnp.int32)
    out_t = _make_tgather(n)(tt, idx)
    return out_t.T.reshape(n, DIM, DIM)


# speedup vs baseline: 1.1924x; 1.1924x over previous
import functools

import jax
import jax.numpy as jnp
from jax import lax
from jax.experimental import pallas as pl
from jax.experimental.pallas import tpu as pltpu
from jax.experimental.pallas import tpu_sc as plsc

UP_TO = 512
DIM = 64
D = DIM * DIM

_NC = 2
_NS = 16
_NW = _NC * _NS

_MW = 2048


def _build_table_kernel(prim_ref, table_ref):
    b0 = prim_ref[0]
    b1 = prim_ref[1]
    table_ref[pl.ds(0, 2)] = prim_ref[pl.ds(0, 2)]
    t2 = jnp.dot(b1, b0, preferred_element_type=jnp.float32)
    table_ref[pl.ds(2, 1)] = t2[None]
    filled = 3
    while filled < UP_TO:
        p0 = (filled + 1) // 2
        s = filled - p0
        parents = table_ref[pl.ds(p0, s)]
        pf = parents.reshape(s * DIM, DIM)
        c1 = jnp.dot(pf, b1, preferred_element_type=jnp.float32)
        c0 = jnp.dot(pf, b0, preferred_element_type=jnp.float32)
        ch = jnp.stack(
            [c1.reshape(s, DIM, DIM), c0.reshape(s, DIM, DIM)], axis=1
        ).reshape(2 * s, DIM, DIM)
        start = 2 * p0 - 1
        cnt = min(2 * s, UP_TO - start)
        table_ref[pl.ds(start, cnt)] = ch[:cnt] if cnt != 2 * s else ch
        filled = start + cnt


def _build_table(primitives):
    return pl.pallas_call(
        _build_table_kernel,
        out_shape=jax.ShapeDtypeStruct((UP_TO, DIM, DIM), jnp.float32),
    )(primitives)


_TC = 128


def _transpose_kernel(in_ref, out_ref):
    out_ref[...] = jnp.transpose(in_ref[...], (1, 0))


def _transpose_table(table2d):
    return pl.pallas_call(
        _transpose_kernel,
        grid=(D // _TC,),
        in_specs=[pl.BlockSpec((UP_TO, _TC), lambda j: (0, j))],
        out_specs=pl.BlockSpec((_TC, UP_TO), lambda j: (j, 0)),
        out_shape=jax.ShapeDtypeStruct((D, UP_TO), jnp.float32),
    )(table2d)


def _make_tgather(n_lookups):
    rpw = D // _NW
    ntr = rpw // 8
    nmc = n_lookups // _MW
    mesh = plsc.VectorSubcoreMesh(core_axis_name="c", subcore_axis_name="s")

    @functools.partial(
        pl.kernel,
        out_type=jax.ShapeDtypeStruct((D, n_lookups), jnp.float32),
        mesh=mesh,
        compiler_params=pltpu.CompilerParams(needs_layout_passes=False),
        scratch_types=[
            pltpu.VMEM((rpw, UP_TO), jnp.float32),
            pltpu.VMEM((n_lookups,), jnp.int32),
            pltpu.VMEM((8, _MW), jnp.float32),
            pltpu.VMEM((8, _MW), jnp.float32),
            pltpu.SemaphoreType.DMA,
            pltpu.SemaphoreType.DMA,
        ],
    )
    def tgather_kernel(tt_hbm, idx_hbm, out_hbm, tab_v, idx_v, buf0, buf1,
                       sem0, sem1):
        wid = lax.axis_index("s") * _NC + lax.axis_index("c")
        r0 = wid * rpw
        pltpu.sync_copy(tt_hbm.at[pl.ds(r0, rpw)], tab_v)
        pltpu.sync_copy(idx_hbm, idx_v)

        bufs = (buf0, buf1)
        sems = (sem0, sem1)
        nblk = ntr * nmc

        @pl.loop(0, nblk, step=2)
        def _(b):
            for k in range(2):
                bb = b + k
                buf = bufs[k]
                sem = sems[k]
                tr = bb // nmc
                mc = bb % nmc
                dst = out_hbm.at[
                    pl.ds(r0 + tr * 8, 8), pl.ds(mc * _MW, _MW)
                ]

                @pl.when(bb >= 2)
                def _():
                    pltpu.make_async_copy(buf, dst, sem).wait()

                m0 = mc * _MW
                rvecs = [
                    jnp.full((16,), tr * 8 + s, jnp.int32) for s in range(8)
                ]
                @plsc.parallel_loop(0, _MW // 16, unroll=4)
                def _(jv):
                    mvec = idx_v[pl.ds(m0 + jv * 16, 16)]
                    for s in range(8):
                        vals = plsc.load_gather(tab_v, [rvecs[s], mvec])
                        buf[s, pl.ds(jv * 16, 16)] = vals

                pltpu.async_copy(buf, dst, sem)

        for k in range(2):
            last = nblk - 2 + k
            tr = last // nmc
            mc = last % nmc
            dst = out_hbm.at[pl.ds(r0 + tr * 8, 8), pl.ds(mc * _MW, _MW)]
            pltpu.make_async_copy(bufs[k], dst, sems[k]).wait()

    return tgather_kernel


def kernel(primitives, node_positions):
    n = node_positions.shape[0]
    table = _build_table(primitives)
    tt = table.reshape(UP_TO, D).T
    idx = (node_positions - 1).astype(jnp.int32)
    out_t = _make_tgather(n)(tt, idx)
    return out_t.T.reshape(n, DIM, DIM)


# speedup vs baseline: 1.2195x; 1.0227x over previous
import functools

import jax
import jax.numpy as jnp
from jax import lax
from jax.experimental import pallas as pl
from jax.experimental.pallas import tpu as pltpu
from jax.experimental.pallas import tpu_sc as plsc

UP_TO = 512
DIM = 64
D = DIM * DIM

_NC = 2
_NS = 16
_NW = _NC * _NS

_MW = 1024
_NBUF = 4


def _build_table_kernel(prim_ref, table_ref):
    b0 = prim_ref[0]
    b1 = prim_ref[1]
    table_ref[pl.ds(0, 2)] = prim_ref[pl.ds(0, 2)]
    t2 = jnp.dot(b1, b0, preferred_element_type=jnp.float32)
    table_ref[pl.ds(2, 1)] = t2[None]
    filled = 3
    while filled < UP_TO:
        p0 = (filled + 1) // 2
        s = filled - p0
        parents = table_ref[pl.ds(p0, s)]
        pf = parents.reshape(s * DIM, DIM)
        c1 = jnp.dot(pf, b1, preferred_element_type=jnp.float32)
        c0 = jnp.dot(pf, b0, preferred_element_type=jnp.float32)
        ch = jnp.stack(
            [c1.reshape(s, DIM, DIM), c0.reshape(s, DIM, DIM)], axis=1
        ).reshape(2 * s, DIM, DIM)
        start = 2 * p0 - 1
        cnt = min(2 * s, UP_TO - start)
        table_ref[pl.ds(start, cnt)] = ch[:cnt] if cnt != 2 * s else ch
        filled = start + cnt


def _build_table(primitives):
    return pl.pallas_call(
        _build_table_kernel,
        out_shape=jax.ShapeDtypeStruct((UP_TO, DIM, DIM), jnp.float32),
    )(primitives)


_TC = 128


def _transpose_kernel(in_ref, out_ref):
    out_ref[...] = jnp.transpose(in_ref[...], (1, 0))


def _transpose_table(table2d):
    return pl.pallas_call(
        _transpose_kernel,
        grid=(D // _TC,),
        in_specs=[pl.BlockSpec((UP_TO, _TC), lambda j: (0, j))],
        out_specs=pl.BlockSpec((_TC, UP_TO), lambda j: (j, 0)),
        out_shape=jax.ShapeDtypeStruct((D, UP_TO), jnp.float32),
    )(table2d)


def _make_tgather(n_lookups):
    rpw = D // _NW
    ntr = rpw // 8
    nmc = n_lookups // _MW
    mesh = plsc.VectorSubcoreMesh(core_axis_name="c", subcore_axis_name="s")

    @functools.partial(
        pl.kernel,
        out_type=jax.ShapeDtypeStruct((D, n_lookups), jnp.float32),
        mesh=mesh,
        compiler_params=pltpu.CompilerParams(needs_layout_passes=False),
        scratch_types=[
            pltpu.VMEM((rpw, UP_TO), jnp.float32),
            pltpu.VMEM((n_lookups,), jnp.int32),
            *[pltpu.VMEM((8, _MW), jnp.float32) for _ in range(_NBUF)],
            *[pltpu.SemaphoreType.DMA for _ in range(_NBUF)],
        ],
    )
    def tgather_kernel(tt_hbm, idx_hbm, out_hbm, tab_v, idx_v, *bufs_sems):
        bufs = bufs_sems[:_NBUF]
        sems = bufs_sems[_NBUF:]
        wid = lax.axis_index("s") * _NC + lax.axis_index("c")
        r0 = wid * rpw
        pltpu.sync_copy(tt_hbm.at[pl.ds(r0, rpw)], tab_v)
        pltpu.sync_copy(idx_hbm, idx_v)

        nblk = ntr * nmc

        @pl.loop(0, nblk, step=_NBUF)
        def _(b):
            for k in range(_NBUF):
                bb = b + k
                buf = bufs[k]
                sem = sems[k]
                tr = bb // nmc
                mc = bb % nmc
                dst = out_hbm.at[
                    pl.ds(r0 + tr * 8, 8), pl.ds(mc * _MW, _MW)
                ]

                @pl.when(bb >= _NBUF)
                def _():
                    pltpu.make_async_copy(buf, dst, sem).wait()

                m0 = mc * _MW
                rvecs = [
                    jnp.full((16,), tr * 8 + s, jnp.int32) for s in range(8)
                ]
                @plsc.parallel_loop(0, _MW // 16, unroll=8)
                def _(jv):
                    mvec = idx_v[pl.ds(m0 + jv * 16, 16)]
                    for s in range(8):
                        vals = plsc.load_gather(tab_v, [rvecs[s], mvec])
                        buf[s, pl.ds(jv * 16, 16)] = vals

                pltpu.async_copy(buf, dst, sem)

        for k in range(_NBUF):
            last = nblk - _NBUF + k
            tr = last // nmc
            mc = last % nmc
            dst = out_hbm.at[pl.ds(r0 + tr * 8, 8), pl.ds(mc * _MW, _MW)]
            pltpu.make_async_copy(bufs[k], dst, sems[k]).wait()

    return tgather_kernel


def kernel(primitives, node_positions):
    n = node_positions.shape[0]
    table = _build_table(primitives)
    tt = table.reshape(UP_TO, D).T
    idx = (node_positions - 1).astype(jnp.int32)
    out_t = _make_tgather(n)(tt, idx)
    return out_t.T.reshape(n, DIM, DIM)


# speedup vs baseline: 1.2748x; 1.0454x over previous
import functools

import jax
import jax.numpy as jnp
from jax import lax
from jax.experimental import pallas as pl
from jax.experimental.pallas import tpu as pltpu
from jax.experimental.pallas import tpu_sc as plsc

UP_TO = 512
DIM = 64
D = DIM * DIM

_NC = 2
_NS = 16
_NW = _NC * _NS

_MW = 1024
_NBUF = 4


def _build_table_kernel(prim_ref, out_ref, table_ref):
    b0 = prim_ref[0]
    b1 = prim_ref[1]
    table_ref[pl.ds(0, 2)] = prim_ref[pl.ds(0, 2)]
    t2 = jnp.dot(b1, b0, preferred_element_type=jnp.float32)
    table_ref[pl.ds(2, 1)] = t2[None]
    filled = 3
    while filled < UP_TO:
        p0 = (filled + 1) // 2
        s = filled - p0
        parents = table_ref[pl.ds(p0, s)]
        pf = parents.reshape(s * DIM, DIM)
        c1 = jnp.dot(pf, b1, preferred_element_type=jnp.float32)
        c0 = jnp.dot(pf, b0, preferred_element_type=jnp.float32)
        ch = jnp.stack(
            [c1.reshape(s, DIM, DIM), c0.reshape(s, DIM, DIM)], axis=1
        ).reshape(2 * s, DIM, DIM)
        start = 2 * p0 - 1
        cnt = min(2 * s, UP_TO - start)
        table_ref[pl.ds(start, cnt)] = ch[:cnt] if cnt != 2 * s else ch
        filled = start + cnt
    t3 = table_ref[...]
    out_ref[...] = jnp.transpose(t3, (1, 2, 0)).reshape(D, UP_TO)


def _build_table_t(primitives):
    return pl.pallas_call(
        _build_table_kernel,
        out_shape=jax.ShapeDtypeStruct((D, UP_TO), jnp.float32),
        scratch_shapes=[pltpu.VMEM((UP_TO, DIM, DIM), jnp.float32)],
    )(primitives)


def _make_tgather(n_lookups):
    rpw = D // _NW
    ntr = rpw // 8
    nmc = n_lookups // _MW
    mesh = plsc.VectorSubcoreMesh(core_axis_name="c", subcore_axis_name="s")

    @functools.partial(
        pl.kernel,
        out_type=jax.ShapeDtypeStruct((D, n_lookups), jnp.float32),
        mesh=mesh,
        compiler_params=pltpu.CompilerParams(needs_layout_passes=False),
        scratch_types=[
            pltpu.VMEM((rpw, UP_TO), jnp.float32),
            pltpu.VMEM((n_lookups,), jnp.int32),
            *[pltpu.VMEM((8, _MW), jnp.float32) for _ in range(_NBUF)],
            *[pltpu.SemaphoreType.DMA for _ in range(_NBUF)],
        ],
    )
    def tgather_kernel(tt_hbm, idx_hbm, out_hbm, tab_v, idx_v, *bufs_sems):
        bufs = bufs_sems[:_NBUF]
        sems = bufs_sems[_NBUF:]
        wid = lax.axis_index("s") * _NC + lax.axis_index("c")
        r0 = wid * rpw
        pltpu.sync_copy(tt_hbm.at[pl.ds(r0, rpw)], tab_v)
        pltpu.sync_copy(idx_hbm, idx_v)

        nblk = ntr * nmc

        @pl.loop(0, nblk, step=_NBUF)
        def _(b):
            for k in range(_NBUF):
                bb = b + k
                buf = bufs[k]
                sem = sems[k]
                tr = bb // nmc
                mc = bb % nmc
                dst = out_hbm.at[
                    pl.ds(r0 + tr * 8, 8), pl.ds(mc * _MW, _MW)
                ]

                @pl.when(bb >= _NBUF)
                def _():
                    pltpu.make_async_copy(buf, dst, sem).wait()

                m0 = mc * _MW
                rvecs = [
                    jnp.full((16,), tr * 8 + s, jnp.int32) for s in range(8)
                ]
                @plsc.parallel_loop(0, _MW // 16, unroll=8)
                def _(jv):
                    mvec = idx_v[pl.ds(m0 + jv * 16, 16)]
                    for s in range(8):
                        vals = plsc.load_gather(tab_v, [rvecs[s], mvec])
                        buf[s, pl.ds(jv * 16, 16)] = vals

                pltpu.async_copy(buf, dst, sem)

        for k in range(_NBUF):
            last = nblk - _NBUF + k
            tr = last // nmc
            mc = last % nmc
            dst = out_hbm.at[pl.ds(r0 + tr * 8, 8), pl.ds(mc * _MW, _MW)]
            pltpu.make_async_copy(bufs[k], dst, sems[k]).wait()

    return tgather_kernel


def kernel(primitives, node_positions):
    n = node_positions.shape[0]
    tt = _build_table_t(primitives)
    idx = (node_positions - 1).astype(jnp.int32)
    out_t = _make_tgather(n)(tt, idx)
    return out_t.T.reshape(n, DIM, DIM)


# speedup vs baseline: 1.2772x; 1.0018x over previous
import functools

import jax
import jax.numpy as jnp
from jax import lax
from jax.experimental import pallas as pl
from jax.experimental.pallas import tpu as pltpu
from jax.experimental.pallas import tpu_sc as plsc

UP_TO = 512
DIM = 64
D = DIM * DIM

_NC = 2
_NS = 16
_NW = _NC * _NS

_MW = 1024
_NBUF = 4


def _build_table_kernel(prim_ref, out_ref, table_ref, tt_ref, sem):
    b0 = prim_ref[0]
    b1 = prim_ref[1]
    table_ref[pl.ds(0, 2)] = prim_ref[pl.ds(0, 2)]
    t2 = jnp.dot(b1, b0, preferred_element_type=jnp.float32)
    table_ref[pl.ds(2, 1)] = t2[None]
    filled = 3
    while filled < UP_TO:
        p0 = (filled + 1) // 2
        s = filled - p0
        parents = table_ref[pl.ds(p0, s)]
        pf = parents.reshape(s * DIM, DIM)
        c1 = jnp.dot(pf, b1, preferred_element_type=jnp.float32)
        c0 = jnp.dot(pf, b0, preferred_element_type=jnp.float32)
        ch = jnp.stack(
            [c1.reshape(s, DIM, DIM), c0.reshape(s, DIM, DIM)], axis=1
        ).reshape(2 * s, DIM, DIM)
        start = 2 * p0 - 1
        cnt = min(2 * s, UP_TO - start)
        table_ref[pl.ds(start, cnt)] = ch[:cnt] if cnt != 2 * s else ch
        filled = start + cnt
    t3 = table_ref[...]
    tt_ref[...] = jnp.transpose(t3, (1, 2, 0)).reshape(D, UP_TO)
    cp = pltpu.make_async_copy(tt_ref, out_ref, sem)
    cp.start()
    cp.wait()


def _build_table_t(primitives):
    return pl.pallas_call(
        _build_table_kernel,
        out_shape=jax.ShapeDtypeStruct((D, UP_TO), jnp.float32),
        out_specs=pl.BlockSpec(memory_space=pltpu.MemorySpace.HBM),
        compiler_params=pltpu.CompilerParams(
            vmem_limit_bytes=100 * 1024 * 1024
        ),
        scratch_shapes=[
            pltpu.VMEM((UP_TO, DIM, DIM), jnp.float32),
            pltpu.VMEM((D, UP_TO), jnp.float32),
            pltpu.SemaphoreType.DMA,
        ],
    )(primitives)


def _make_tgather(n_lookups):
    rpw = D // _NW
    ntr = rpw // 8
    nmc = n_lookups // _MW
    mesh = plsc.VectorSubcoreMesh(core_axis_name="c", subcore_axis_name="s")

    @functools.partial(
        pl.kernel,
        out_type=jax.ShapeDtypeStruct((D, n_lookups), jnp.float32),
        mesh=mesh,
        compiler_params=pltpu.CompilerParams(needs_layout_passes=False),
        scratch_types=[
            pltpu.VMEM((rpw, UP_TO), jnp.float32),
            pltpu.VMEM((n_lookups,), jnp.int32),
            *[pltpu.VMEM((8, _MW), jnp.float32) for _ in range(_NBUF)],
            *[pltpu.SemaphoreType.DMA for _ in range(_NBUF)],
        ],
    )
    def tgather_kernel(tt_hbm, idx_hbm, out_hbm, tab_v, idx_v, *bufs_sems):
        bufs = bufs_sems[:_NBUF]
        sems = bufs_sems[_NBUF:]
        wid = lax.axis_index("s") * _NC + lax.axis_index("c")
        r0 = wid * rpw
        pltpu.sync_copy(tt_hbm.at[pl.ds(r0, rpw)], tab_v)
        pltpu.sync_copy(idx_hbm, idx_v)

        nblk = ntr * nmc

        @pl.loop(0, nblk, step=_NBUF)
        def _(b):
            for k in range(_NBUF):
                bb = b + k
                buf = bufs[k]
                sem = sems[k]
                tr = bb // nmc
                mc = bb % nmc
                dst = out_hbm.at[
                    pl.ds(r0 + tr * 8, 8), pl.ds(mc * _MW, _MW)
                ]

                @pl.when(bb >= _NBUF)
                def _():
                    pltpu.make_async_copy(buf, dst, sem).wait()

                m0 = mc * _MW
                rvecs = [
                    jnp.full((16,), tr * 8 + s, jnp.int32) for s in range(8)
                ]
                @plsc.parallel_loop(0, _MW // 16, unroll=8)
                def _(jv):
                    mvec = idx_v[pl.ds(m0 + jv * 16, 16)]
                    for s in range(8):
                        vals = plsc.load_gather(tab_v, [rvecs[s], mvec])
                        buf[s, pl.ds(jv * 16, 16)] = vals

                pltpu.async_copy(buf, dst, sem)

        for k in range(_NBUF):
            last = nblk - _NBUF + k
            tr = last // nmc
            mc = last % nmc
            dst = out_hbm.at[pl.ds(r0 + tr * 8, 8), pl.ds(mc * _MW, _MW)]
            pltpu.make_async_copy(bufs[k], dst, sems[k]).wait()

    return tgather_kernel


def kernel(primitives, node_positions):
    n = node_positions.shape[0]
    tt = _build_table_t(primitives)
    idx = (node_positions - 1).astype(jnp.int32)
    out_t = _make_tgather(n)(tt, idx)
    return out_t.T.reshape(n, DIM, DIM)


# speedup vs baseline: 1.2774x; 1.0001x over previous
import functools

import jax
import jax.numpy as jnp
from jax import lax
from jax.experimental import pallas as pl
from jax.experimental.pallas import tpu as pltpu
from jax.experimental.pallas import tpu_sc as plsc

UP_TO = 512
DIM = 64
D = DIM * DIM

_NC = 2
_NS = 16
_NW = _NC * _NS

_MW = 1024
_NBUF = 4


def _build_table_kernel(prim_ref, out_ref, table_ref, tt_ref, sem):
    b0 = prim_ref[0]
    b1 = prim_ref[1]
    table_ref[pl.ds(0, 2)] = prim_ref[pl.ds(0, 2)]
    t2 = jnp.dot(b1, b0, preferred_element_type=jnp.float32)
    table_ref[pl.ds(2, 1)] = t2[None]
    filled = 3
    while filled < UP_TO:
        p0 = (filled + 1) // 2
        s = filled - p0
        parents = table_ref[pl.ds(p0, s)]
        pf = parents.reshape(s * DIM, DIM)
        c1 = jnp.dot(pf, b1, preferred_element_type=jnp.float32)
        c0 = jnp.dot(pf, b0, preferred_element_type=jnp.float32)
        ch = jnp.stack(
            [c1.reshape(s, DIM, DIM), c0.reshape(s, DIM, DIM)], axis=1
        ).reshape(2 * s, DIM, DIM)
        start = 2 * p0 - 1
        cnt = min(2 * s, UP_TO - start)
        table_ref[pl.ds(start, cnt)] = ch[:cnt] if cnt != 2 * s else ch
        filled = start + cnt
    t3 = table_ref[...]
    tt_ref[...] = jnp.transpose(t3, (1, 2, 0)).reshape(D, UP_TO)
    cp = pltpu.make_async_copy(tt_ref, out_ref, sem)
    cp.start()
    cp.wait()


def _build_table_t(primitives):
    return pl.pallas_call(
        _build_table_kernel,
        out_shape=jax.ShapeDtypeStruct((D, UP_TO), jnp.float32),
        out_specs=pl.BlockSpec(memory_space=pltpu.MemorySpace.HBM),
        compiler_params=pltpu.CompilerParams(
            vmem_limit_bytes=100 * 1024 * 1024
        ),
        scratch_shapes=[
            pltpu.VMEM((UP_TO, DIM, DIM), jnp.float32),
            pltpu.VMEM((D, UP_TO), jnp.float32),
            pltpu.SemaphoreType.DMA,
        ],
    )(primitives)


def _make_tgather(n_lookups):
    rpw = D // _NW
    ntr = rpw // 8
    nmc = n_lookups // _MW
    mesh = plsc.VectorSubcoreMesh(core_axis_name="c", subcore_axis_name="s")

    ntc = _MW // 128

    @functools.partial(
        pl.kernel,
        out_type=jax.ShapeDtypeStruct(
            (D // 8, n_lookups // 128, 8, 128), jnp.float32
        ),
        mesh=mesh,
        compiler_params=pltpu.CompilerParams(needs_layout_passes=False),
        scratch_types=[
            pltpu.VMEM((rpw, UP_TO), jnp.float32),
            pltpu.VMEM((n_lookups,), jnp.int32),
            *[pltpu.VMEM((ntc, 8, 128), jnp.float32) for _ in range(_NBUF)],
            *[pltpu.SemaphoreType.DMA for _ in range(_NBUF)],
        ],
    )
    def tgather_kernel(tt_hbm, idx_hbm, out_hbm, tab_v, idx_v, *bufs_sems):
        bufs = bufs_sems[:_NBUF]
        sems = bufs_sems[_NBUF:]
        wid = lax.axis_index("s") * _NC + lax.axis_index("c")
        r0 = wid * rpw
        pltpu.sync_copy(tt_hbm.at[pl.ds(r0, rpw)], tab_v)
        pltpu.sync_copy(idx_hbm, idx_v)

        nblk = ntr * nmc

        @pl.loop(0, nblk, step=_NBUF)
        def _(b):
            for k in range(_NBUF):
                bb = b + k
                buf = bufs[k]
                sem = sems[k]
                tr = bb // nmc
                mc = bb % nmc
                dst = out_hbm.at[
                    wid * ntr + tr, pl.ds(mc * ntc, ntc)
                ]

                @pl.when(bb >= _NBUF)
                def _():
                    pltpu.make_async_copy(buf, dst, sem).wait()

                m0 = mc * _MW
                rvecs = [
                    jnp.full((16,), tr * 8 + s, jnp.int32) for s in range(8)
                ]
                @plsc.parallel_loop(0, _MW // 16, unroll=8)
                def _(jv):
                    mvec = idx_v[pl.ds(m0 + jv * 16, 16)]
                    for s in range(8):
                        vals = plsc.load_gather(tab_v, [rvecs[s], mvec])
                        buf[jv // 8, s, pl.ds((jv % 8) * 16, 16)] = vals

                pltpu.async_copy(buf, dst, sem)

        for k in range(_NBUF):
            last = nblk - _NBUF + k
            tr = last // nmc
            mc = last % nmc
            dst = out_hbm.at[wid * ntr + tr, pl.ds(mc * ntc, ntc)]
            pltpu.make_async_copy(bufs[k], dst, sems[k]).wait()

    return tgather_kernel


def kernel(primitives, node_positions):
    n = node_positions.shape[0]
    tt = _build_table_t(primitives)
    idx = (node_positions - 1).astype(jnp.int32)
    out4 = _make_tgather(n)(tt, idx)
    out_t = out4.transpose(0, 2, 1, 3).reshape(D, n)
    return out_t.T.reshape(n, DIM, DIM)
